# Initial kernel scaffold; baseline (speedup 1.0000x reference)
import jax
import jax.numpy as jnp
from jax.experimental import pallas as pl


def kernel(x, edge_index, edge_attr, t, s, W_embed, b_embed, Wt, bt, Ws, bs, g1, b1, Wg, a_src, a_dst, We, a_e, bg, W_dec, b_dec):
    raise NotImplementedError("write your pallas kernel here")


# speedup vs baseline: 14.8745x; 14.8745x over previous
import jax
import jax.numpy as jnp
from jax import lax
from jax.experimental import pallas as pl
from jax.experimental.pallas import tpu as pltpu
from jax.experimental.pallas import tpu_sc as plsc

N = 10000
E = 320000
HID = 128
HH = 64
NL = 5
NPAD = 10112
NS = 16
CHUNKS = 320
CH = 64
WCH = 16
EP = NS * CHUNKS * CH
RPT = NPAD // NS
F32 = jnp.float32


def _prologue_body(xf_ref, t_ref, s_ref, We_ref, be_ref, Wt_ref, bt_ref,
                   Ws_ref, bs_ref, g1_ref, b1_ref, Wg0_ref, asd_ref,
                   hw_ref, sd_ref):
    h = jnp.dot(xf_ref[...], We_ref[...], preferred_element_type=F32) + be_ref[...]
    h = h + jax.nn.relu(jnp.dot(t_ref[...], Wt_ref[...], preferred_element_type=F32) + bt_ref[...])
    h = h + jax.nn.relu(jnp.dot(s_ref[...], Ws_ref[...], preferred_element_type=F32) + bs_ref[...])
    m = jnp.mean(h, axis=-1, keepdims=True)
    v = jnp.mean((h - m) ** 2, axis=-1, keepdims=True)
    h = (h - m) / jnp.sqrt(v + 1e-5) * g1_ref[...] + b1_ref[...]
    h = jax.nn.relu(h)
    hw = jnp.dot(h, Wg0_ref[...], preferred_element_type=F32)
    hw_ref[...] = hw
    sd_ref[...] = jnp.dot(hw, asd_ref[...], preferred_element_type=F32)


def _prologue_call(xf, t, s, We, be, Wt, bt, Ws, bs, g1, b1, Wg0, asd):
    full = lambda a: pl.BlockSpec(a.shape, lambda r: tuple(0 for _ in a.shape))
    return pl.pallas_call(
        _prologue_body,
        grid=(10,),
        in_specs=[
            pl.BlockSpec((1000, 24), lambda r: (r, 0)),
            pl.BlockSpec((1000, 4), lambda r: (r, 0)),
            pl.BlockSpec((1000, 6), lambda r: (r, 0)),
            full(We), full(be), full(Wt), full(bt), full(Ws), full(bs),
            full(g1), full(b1), full(Wg0), full(asd),
        ],
        out_specs=[
            pl.BlockSpec((1000, HID), lambda r: (r, 0)),
            pl.BlockSpec((1000, 2), lambda r: (r, 0)),
        ],
        out_shape=[
            jax.ShapeDtypeStruct((N, HID), F32),
            jax.ShapeDtypeStruct((N, 2), F32),
        ],
    )(xf, t, s, We, be, Wt, bt, Ws, bs, g1, b1, Wg0, asd)


def _edge_prologue_body(eaT_ref, We3_ref, ae3_ref, eal_ref, wea_ref, easum_ref):
    j = pl.program_id(0)
    ea_blk = eaT_ref[...]
    for l in range(NL):
        wl = jnp.dot(We3_ref[l], ae3_ref[l], preferred_element_type=F32)
        wea_ref[:, l:l + 1] = wl
        eal_ref[l:l + 1, :] = lax.dot_general(
            wl, ea_blk, (((0,), (0,)), ((), ())), preferred_element_type=F32)

    @pl.when(j == 0)
    def _():
        easum_ref[...] = jnp.zeros_like(easum_ref)

    easum_ref[...] += jnp.sum(ea_blk, axis=1, keepdims=True)


def _edge_prologue_call(eaT, We3, ae3):
    BLK = 4096
    full = lambda a: pl.BlockSpec(a.shape, lambda r: tuple(0 for _ in a.shape))
    return pl.pallas_call(
        _edge_prologue_body,
        grid=(EP // BLK,),
        in_specs=[
            pl.BlockSpec((4, BLK), lambda r: (0, r)),
            full(We3), full(ae3),
        ],
        out_specs=[
            pl.BlockSpec((NL, BLK), lambda r: (0, r)),
            pl.BlockSpec((4, NL), lambda r: (0, 0)),
            pl.BlockSpec((4, 1), lambda r: (0, 0)),
        ],
        out_shape=[
            jax.ShapeDtypeStruct((NL, EP), F32),
            jax.ShapeDtypeStruct((4, NL), F32),
            jax.ShapeDtypeStruct((4, 1), F32),
        ],
    )(eaT, We3, ae3)


def _self_weight(hwp, asum_ref, easum_ref, weal_ref):
    eaself = jnp.sum(easum_ref[...] * weal_ref[...]) * (1.0 / E)
    a = jnp.dot(hwp, asum_ref[...], preferred_element_type=F32) + eaself
    a = jnp.where(a >= 0.0, a, a * 0.2)
    return jnp.exp(a)


def _combine_h(num_ref, den_ref, hwp, ws, bg_ref):
    num = jnp.concatenate([num_ref[0], num_ref[1]], axis=-1) + ws * hwp
    den = den_ref[0, :, 0:1] + ws + 1e-16
    return jax.nn.relu(num / den + bg_ref[...])


def _combine_mid_body(num_ref, den_ref, hwp_ref, easum_ref, weal_ref, asum_ref,
                      bg_ref, Wn_ref, asdn_ref, hw_ref, sd_ref):
    hwp = hwp_ref[...]
    ws = _self_weight(hwp, asum_ref, easum_ref, weal_ref)
    h = _combine_h(num_ref, den_ref, hwp, ws, bg_ref)
    hw = jnp.dot(h, Wn_ref[...], preferred_element_type=F32)
    hw_ref[...] = hw
    sd_ref[...] = jnp.dot(hw, asdn_ref[...], preferred_element_type=F32)


def _combine_final_body(num_ref, den_ref, hwp_ref, easum_ref, weal_ref, asum_ref,
                        bg_ref, Wd_ref, bd_ref, out_ref):
    hwp = hwp_ref[...]
    ws = _self_weight(hwp, asum_ref, easum_ref, weal_ref)
    h = _combine_h(num_ref, den_ref, hwp, ws, bg_ref)
    out_ref[...] = jnp.dot(h, Wd_ref[...], preferred_element_type=F32) + bd_ref[...]


def _combine_in_specs(full, extra):
    return [
        pl.BlockSpec((2, 1000, HH), lambda r: (0, r, 0)),
        pl.BlockSpec((2, 1000, 16), lambda r: (0, r, 0)),
        pl.BlockSpec((1000, HID), lambda r: (r, 0)),
    ] + [full(a) for a in extra]


def _combine_mid_call(num, den, hwp, easum, weal, asum, bg, Wn, asdn):
    full = lambda a: pl.BlockSpec(a.shape, lambda r: tuple(0 for _ in a.shape))
    return pl.pallas_call(
        _combine_mid_body,
        grid=(10,),
        in_specs=_combine_in_specs(full, [easum, weal, asum, bg, Wn, asdn]),
        out_specs=[
            pl.BlockSpec((1000, HID), lambda r: (r, 0)),
            pl.BlockSpec((1000, 2), lambda r: (r, 0)),
        ],
        out_shape=[
            jax.ShapeDtypeStruct((N, HID), F32),
            jax.ShapeDtypeStruct((N, 2), F32),
        ],
    )(num, den, hwp, easum, weal, asum, bg, Wn, asdn)


def _combine_final_call(num, den, hwp, easum, weal, asum, bg, Wd, bd):
    full = lambda a: pl.BlockSpec(a.shape, lambda r: tuple(0 for _ in a.shape))
    return pl.pallas_call(
        _combine_final_body,
        grid=(10,),
        in_specs=_combine_in_specs(full, [easum, weal, asum, bg, Wd, bd]),
        out_specs=pl.BlockSpec((1000, 24), lambda r: (r, 0)),
        out_shape=jax.ShapeDtypeStruct((N, 24), F32),
    )(num, den, hwp, easum, weal, asum, bg, Wd, bd)


def _sc_edge_body(hw_h, sd_h, src_h, dst_h, ea_h, z64_h, z16_h,
                  num_o, den_o,
                  sd_v, src_w, dst_w, ea_w, wch, rg0, rg1, rs, wrep,
                  num_sh, den_sh, gsem0, gsem1):
    c = lax.axis_index("c")
    s = lax.axis_index("s")

    pltpu.sync_copy(sd_h, sd_v)
    base = s * RPT
    pltpu.sync_copy(z64_h.at[pl.ds(base, RPT)], num_sh.at[pl.ds(base, RPT)])
    pltpu.sync_copy(z16_h.at[pl.ds(base, RPT)], den_sh.at[pl.ds(base, RPT)])
    plsc.subcore_barrier()

    fbase = c * HH
    oi = jnp.full((16,), 1, jnp.int32)
    nclamp = jnp.full((16,), N - 1, jnp.int32)

    def do_chunk(k, rg, gsem):
        pltpu.make_async_copy(hw_h.at[src_w.at[k]], rg, gsem).wait()

        def agrp(q, carry):
            sl = pl.ds(q * 16, 16)
            sv = src_w[k, sl]
            dv = jnp.minimum(dst_w[k, sl], nclamp)
            a = (plsc.load_gather(sd_v, [sv + sv])
                 + plsc.load_gather(sd_v, [dv + dv + oi])
                 + ea_w[k, sl])
            a = jnp.where(a >= 0.0, a, a * 0.2)
            wch[sl] = jnp.exp(a)
            return carry

        lax.fori_loop(0, CH // 16, agrp, 0)

        def srow(e, carry):
            we = plsc.load_gather(wch, [jnp.full((16,), e, jnp.int32)])
            wrep[e, :] = we
            for q in range(HH // 16):
                rs[e, pl.ds(q * 16, 16)] = rg[e, pl.ds(fbase + q * 16, 16)] * we
            return carry

        lax.fori_loop(0, CH, srow, 0)

        @pl.when(k + 2 < WCH)
        def _():
            pltpu.make_async_copy(hw_h.at[src_w.at[k + 2]], rg, gsem).start()

        pltpu.sync_copy(rs, num_sh.at[dst_w.at[k]], add=True)
        pltpu.sync_copy(wrep, den_sh.at[dst_w.at[k]], add=True)

    def window(wi, carry):
        woff = wi * WCH
        pltpu.sync_copy(src_h.at[s, pl.ds(woff, WCH)], src_w)
        pltpu.sync_copy(dst_h.at[s, pl.ds(woff, WCH)], dst_w)
        pltpu.sync_copy(ea_h.at[s, pl.ds(woff, WCH)], ea_w)
        pltpu.make_async_copy(hw_h.at[src_w.at[0]], rg0, gsem0).start()
        pltpu.make_async_copy(hw_h.at[src_w.at[1]], rg1, gsem1).start()

        def pair(p, carry2):
            do_chunk(2 * p, rg0, gsem0)
            do_chunk(2 * p + 1, rg1, gsem1)
            return carry2

        return lax.fori_loop(0, WCH // 2, pair, carry)

    lax.fori_loop(0, CHUNKS // WCH, window, 0)
    plsc.subcore_barrier()

    pltpu.sync_copy(num_sh.at[pl.ds(base, RPT)], num_o.at[c, pl.ds(base, RPT)])
    pltpu.sync_copy(den_sh.at[pl.ds(base, RPT)], den_o.at[c, pl.ds(base, RPT)])


def _make_sc_edge(interpret=False):
    return pl.kernel(
        _sc_edge_body,
        out_type=[jax.ShapeDtypeStruct((2, NPAD, HH), F32),
                  jax.ShapeDtypeStruct((2, NPAD, 16), F32)],
        mesh=plsc.VectorSubcoreMesh(core_axis_name="c", subcore_axis_name="s",
                                    num_cores=2, num_subcores=16),
        compiler_params=pltpu.CompilerParams(needs_layout_passes=False,
                                             use_tc_tiling_on_sc=False),
        scratch_types=[
            pltpu.VMEM((2 * N,), F32),
            pltpu.VMEM((WCH, CH), jnp.int32),
            pltpu.VMEM((WCH, CH), jnp.int32),
            pltpu.VMEM((WCH, CH), F32),
            pltpu.VMEM((CH,), F32),
            pltpu.VMEM((CH, HID), F32),
            pltpu.VMEM((CH, HID), F32),
            pltpu.VMEM((CH, HH), F32),
            pltpu.VMEM((CH, 16), F32),
            pltpu.VMEM_SHARED((NPAD, HH), F32),
            pltpu.VMEM_SHARED((NPAD, 16), F32),
            pltpu.SemaphoreType.DMA,
            pltpu.SemaphoreType.DMA,
        ],
        interpret=interpret,
    )


_sc_edge_cache = {}


def _sc_edge(*args):
    if "k" not in _sc_edge_cache:
        _sc_edge_cache["k"] = _make_sc_edge()
    return _sc_edge_cache["k"](*args)


def kernel(x, edge_index, edge_attr, t, s, W_embed, b_embed, Wt, bt, Ws, bs,
           g1, b1, Wg, a_src, a_dst, We, a_e, bg, W_dec, b_dec):
    xf = x.reshape(N, 24)
    src = edge_index[0]
    dst = edge_index[1]
    npe = EP - E
    pad_src = jnp.arange(npe, dtype=jnp.int32) % N
    pad_dst = N + (jnp.arange(npe, dtype=jnp.int32) % (NPAD - N))
    srcP = jnp.concatenate([src, pad_src]).reshape(NS, CHUNKS, CH)
    dstP = jnp.concatenate([dst, pad_dst]).reshape(NS, CHUNKS, CH)
    eaT = jnp.concatenate([edge_attr.T, jnp.zeros((4, npe), F32)], axis=1)
    z64 = jnp.zeros((NPAD, HH), F32)
    z16 = jnp.zeros((NPAD, 16), F32)

    hw, sd = _prologue_call(
        xf, t, s, W_embed, b_embed.reshape(1, HID), Wt, bt.reshape(1, HID),
        Ws, bs.reshape(1, HID), g1.reshape(1, HID), b1.reshape(1, HID),
        Wg[0], jnp.stack([a_src[0], a_dst[0]], axis=1))
    eal, wea, easum = _edge_prologue_call(eaT, We, a_e.reshape(NL, HID, 1))
    ealP = eal.reshape(NL, NS, CHUNKS, CH)

    for i in range(NL):
        num, den = _sc_edge(hw, sd.reshape(2 * N), srcP, dstP, ealP[i],
                            z64, z16)
        weal = wea[:, i:i + 1]
        asum = (a_src[i] + a_dst[i]).reshape(HID, 1)
        if i < NL - 1:
            hw, sd = _combine_mid_call(
                num, den, hw, easum, weal, asum, bg[i].reshape(1, HID),
                Wg[i + 1], jnp.stack([a_src[i + 1], a_dst[i + 1]], axis=1))
        else:
            out = _combine_final_call(
                num, den, hw, easum, weal, asum, bg[i].reshape(1, HID),
                W_dec, b_dec.reshape(1, 24))
    return out.reshape(N, 2, 12)


# speedup vs baseline: 15.2245x; 1.0235x over previous
import jax
import jax.numpy as jnp
from jax import lax
from jax.experimental import pallas as pl
from jax.experimental.pallas import tpu as pltpu
from jax.experimental.pallas import tpu_sc as plsc

N = 10000
E = 320000
HID = 128
HH = 64
NL = 5
NPAD = 10112
NS = 16
CHUNKS = 320
CH = 64
WCH = 16
EP = NS * CHUNKS * CH
RPT = NPAD // NS
F32 = jnp.float32


def _prologue_body(xf_ref, t_ref, s_ref, We_ref, be_ref, Wt_ref, bt_ref,
                   Ws_ref, bs_ref, g1_ref, b1_ref, Wg0_ref, asd_ref,
                   hw_ref, sd_ref):
    h = jnp.dot(xf_ref[...], We_ref[...], preferred_element_type=F32) + be_ref[...]
    h = h + jax.nn.relu(jnp.dot(t_ref[...], Wt_ref[...], preferred_element_type=F32) + bt_ref[...])
    h = h + jax.nn.relu(jnp.dot(s_ref[...], Ws_ref[...], preferred_element_type=F32) + bs_ref[...])
    m = jnp.mean(h, axis=-1, keepdims=True)
    v = jnp.mean((h - m) ** 2, axis=-1, keepdims=True)
    h = (h - m) / jnp.sqrt(v + 1e-5) * g1_ref[...] + b1_ref[...]
    h = jax.nn.relu(h)
    hw = jnp.dot(h, Wg0_ref[...], preferred_element_type=F32)
    hw_ref[...] = hw
    sd_ref[...] = jnp.dot(hw, asd_ref[...], preferred_element_type=F32)


def _prologue_call(xf, t, s, We, be, Wt, bt, Ws, bs, g1, b1, Wg0, asd):
    full = lambda a: pl.BlockSpec(a.shape, lambda r: tuple(0 for _ in a.shape))
    return pl.pallas_call(
        _prologue_body,
        grid=(10,),
        in_specs=[
            pl.BlockSpec((1000, 24), lambda r: (r, 0)),
            pl.BlockSpec((1000, 4), lambda r: (r, 0)),
            pl.BlockSpec((1000, 6), lambda r: (r, 0)),
            full(We), full(be), full(Wt), full(bt), full(Ws), full(bs),
            full(g1), full(b1), full(Wg0), full(asd),
        ],
        out_specs=[
            pl.BlockSpec((1000, HID), lambda r: (r, 0)),
            pl.BlockSpec((1000, 2), lambda r: (r, 0)),
        ],
        out_shape=[
            jax.ShapeDtypeStruct((N, HID), F32),
            jax.ShapeDtypeStruct((N, 2), F32),
        ],
    )(xf, t, s, We, be, Wt, bt, Ws, bs, g1, b1, Wg0, asd)


def _edge_prologue_body(eaT_ref, We3_ref, ae3_ref, eal_ref, wea_ref, easum_ref):
    j = pl.program_id(0)
    ea_blk = eaT_ref[...]
    for l in range(NL):
        wl = jnp.dot(We3_ref[l], ae3_ref[l], preferred_element_type=F32)
        wea_ref[:, l:l + 1] = wl
        eal_ref[l:l + 1, :] = lax.dot_general(
            wl, ea_blk, (((0,), (0,)), ((), ())), preferred_element_type=F32)

    @pl.when(j == 0)
    def _():
        easum_ref[...] = jnp.zeros_like(easum_ref)

    easum_ref[...] += jnp.sum(ea_blk, axis=1, keepdims=True)


def _edge_prologue_call(eaT, We3, ae3):
    BLK = 4096
    full = lambda a: pl.BlockSpec(a.shape, lambda r: tuple(0 for _ in a.shape))
    return pl.pallas_call(
        _edge_prologue_body,
        grid=(EP // BLK,),
        in_specs=[
            pl.BlockSpec((4, BLK), lambda r: (0, r)),
            full(We3), full(ae3),
        ],
        out_specs=[
            pl.BlockSpec((NL, BLK), lambda r: (0, r)),
            pl.BlockSpec((4, NL), lambda r: (0, 0)),
            pl.BlockSpec((4, 1), lambda r: (0, 0)),
        ],
        out_shape=[
            jax.ShapeDtypeStruct((NL, EP), F32),
            jax.ShapeDtypeStruct((4, NL), F32),
            jax.ShapeDtypeStruct((4, 1), F32),
        ],
    )(eaT, We3, ae3)


def _self_weight(hwp, asum_ref, easum_ref, weal_ref):
    eaself = jnp.sum(easum_ref[...] * weal_ref[...]) * (1.0 / E)
    a = jnp.dot(hwp, asum_ref[...], preferred_element_type=F32) + eaself
    a = jnp.where(a >= 0.0, a, a * 0.2)
    return jnp.exp(a)


def _combine_h(num_ref, den_ref, hwp, ws, bg_ref):
    num = jnp.concatenate([num_ref[0], num_ref[1]], axis=-1) + ws * hwp
    den = den_ref[0, :, 0:1] + ws + 1e-16
    return jax.nn.relu(num / den + bg_ref[...])


def _combine_mid_body(num_ref, den_ref, hwp_ref, easum_ref, weal_ref, asum_ref,
                      bg_ref, Wn_ref, asdn_ref, hw_ref, sd_ref):
    hwp = hwp_ref[...]
    ws = _self_weight(hwp, asum_ref, easum_ref, weal_ref)
    h = _combine_h(num_ref, den_ref, hwp, ws, bg_ref)
    hw = jnp.dot(h, Wn_ref[...], preferred_element_type=F32)
    hw_ref[...] = hw
    sd_ref[...] = jnp.dot(hw, asdn_ref[...], preferred_element_type=F32)


def _combine_final_body(num_ref, den_ref, hwp_ref, easum_ref, weal_ref, asum_ref,
                        bg_ref, Wd_ref, bd_ref, out_ref):
    hwp = hwp_ref[...]
    ws = _self_weight(hwp, asum_ref, easum_ref, weal_ref)
    h = _combine_h(num_ref, den_ref, hwp, ws, bg_ref)
    out_ref[...] = jnp.dot(h, Wd_ref[...], preferred_element_type=F32) + bd_ref[...]


def _combine_in_specs(full, extra):
    return [
        pl.BlockSpec((2, 1000, HH), lambda r: (0, r, 0)),
        pl.BlockSpec((2, 1000, 16), lambda r: (0, r, 0)),
        pl.BlockSpec((1000, HID), lambda r: (r, 0)),
    ] + [full(a) for a in extra]


def _combine_mid_call(num, den, hwp, easum, weal, asum, bg, Wn, asdn):
    full = lambda a: pl.BlockSpec(a.shape, lambda r: tuple(0 for _ in a.shape))
    return pl.pallas_call(
        _combine_mid_body,
        grid=(10,),
        in_specs=_combine_in_specs(full, [easum, weal, asum, bg, Wn, asdn]),
        out_specs=[
            pl.BlockSpec((1000, HID), lambda r: (r, 0)),
            pl.BlockSpec((1000, 2), lambda r: (r, 0)),
        ],
        out_shape=[
            jax.ShapeDtypeStruct((N, HID), F32),
            jax.ShapeDtypeStruct((N, 2), F32),
        ],
    )(num, den, hwp, easum, weal, asum, bg, Wn, asdn)


def _combine_final_call(num, den, hwp, easum, weal, asum, bg, Wd, bd):
    full = lambda a: pl.BlockSpec(a.shape, lambda r: tuple(0 for _ in a.shape))
    return pl.pallas_call(
        _combine_final_body,
        grid=(10,),
        in_specs=_combine_in_specs(full, [easum, weal, asum, bg, Wd, bd]),
        out_specs=pl.BlockSpec((1000, 24), lambda r: (r, 0)),
        out_shape=jax.ShapeDtypeStruct((N, 24), F32),
    )(num, den, hwp, easum, weal, asum, bg, Wd, bd)


def _sc_edge_body(hw_h, sd_h, src_h, dst_h, ea_h, z64_h, z16_h,
                  num_o, den_o,
                  sd_v, src_w, dst_w, ea_w, wch, rg0, rg1, rs, wrep,
                  num_sh, den_sh, gsem0, gsem1):
    c = lax.axis_index("c")
    s = lax.axis_index("s")

    pltpu.sync_copy(sd_h, sd_v)
    base = s * RPT
    pltpu.sync_copy(z64_h.at[pl.ds(base, RPT)], num_sh.at[pl.ds(base, RPT)])
    pltpu.sync_copy(z16_h.at[pl.ds(base, RPT)], den_sh.at[pl.ds(base, RPT)])
    plsc.subcore_barrier()

    fbase = c * HH
    oi = jnp.full((16,), 1, jnp.int32)
    nclamp = jnp.full((16,), N - 1, jnp.int32)

    def do_chunk(k, rg, gsem):
        pltpu.make_async_copy(hw_h.at[src_w.at[k]], rg, gsem).wait()

        def agrp(q, carry):
            sl = pl.ds(q * 16, 16)
            sv = src_w[k, sl]
            dv = jnp.minimum(dst_w[k, sl], nclamp)
            a = (plsc.load_gather(sd_v, [sv + sv])
                 + plsc.load_gather(sd_v, [dv + dv + oi])
                 + ea_w[k, sl])
            a = jnp.where(a >= 0.0, a, a * 0.2)
            wch[sl] = jnp.exp(a)
            return carry

        lax.fori_loop(0, CH // 16, agrp, 0)

        def srow(e4, carry):
            for u in range(4):
                e = e4 * 4 + u
                we = plsc.load_gather(wch, [jnp.full((16,), e, jnp.int32)])
                wrep[e, :] = we
                for q in range(HH // 16):
                    rs[e, pl.ds(q * 16, 16)] = rg[e, pl.ds(fbase + q * 16, 16)] * we
            return carry

        lax.fori_loop(0, CH // 4, srow, 0)

        @pl.when(k + 2 < WCH)
        def _():
            pltpu.make_async_copy(hw_h.at[src_w.at[k + 2]], rg, gsem).start()

        pltpu.sync_copy(rs, num_sh.at[dst_w.at[k]], add=True)
        pltpu.sync_copy(wrep, den_sh.at[dst_w.at[k]], add=True)

    def window(wi, carry):
        woff = wi * WCH
        pltpu.sync_copy(src_h.at[s, pl.ds(woff, WCH)], src_w)
        pltpu.sync_copy(dst_h.at[s, pl.ds(woff, WCH)], dst_w)
        pltpu.sync_copy(ea_h.at[s, pl.ds(woff, WCH)], ea_w)
        pltpu.make_async_copy(hw_h.at[src_w.at[0]], rg0, gsem0).start()
        pltpu.make_async_copy(hw_h.at[src_w.at[1]], rg1, gsem1).start()

        def pair(p, carry2):
            do_chunk(2 * p, rg0, gsem0)
            do_chunk(2 * p + 1, rg1, gsem1)
            return carry2

        return lax.fori_loop(0, WCH // 2, pair, carry)

    lax.fori_loop(0, CHUNKS // WCH, window, 0)
    plsc.subcore_barrier()

    pltpu.sync_copy(num_sh.at[pl.ds(base, RPT)], num_o.at[c, pl.ds(base, RPT)])
    pltpu.sync_copy(den_sh.at[pl.ds(base, RPT)], den_o.at[c, pl.ds(base, RPT)])


def _make_sc_edge(interpret=False):
    return pl.kernel(
        _sc_edge_body,
        out_type=[jax.ShapeDtypeStruct((2, NPAD, HH), F32),
                  jax.ShapeDtypeStruct((2, NPAD, 16), F32)],
        mesh=plsc.VectorSubcoreMesh(core_axis_name="c", subcore_axis_name="s",
                                    num_cores=2, num_subcores=16),
        compiler_params=pltpu.CompilerParams(needs_layout_passes=False,
                                             use_tc_tiling_on_sc=False),
        scratch_types=[
            pltpu.VMEM((2 * N,), F32),
            pltpu.VMEM((WCH, CH), jnp.int32),
            pltpu.VMEM((WCH, CH), jnp.int32),
            pltpu.VMEM((WCH, CH), F32),
            pltpu.VMEM((CH,), F32),
            pltpu.VMEM((CH, HID), F32),
            pltpu.VMEM((CH, HID), F32),
            pltpu.VMEM((CH, HH), F32),
            pltpu.VMEM((CH, 16), F32),
            pltpu.VMEM_SHARED((NPAD, HH), F32),
            pltpu.VMEM_SHARED((NPAD, 16), F32),
            pltpu.SemaphoreType.DMA,
            pltpu.SemaphoreType.DMA,
        ],
        interpret=interpret,
    )


_sc_edge_cache = {}


def _sc_edge(*args):
    if "k" not in _sc_edge_cache:
        _sc_edge_cache["k"] = _make_sc_edge()
    return _sc_edge_cache["k"](*args)


def kernel(x, edge_index, edge_attr, t, s, W_embed, b_embed, Wt, bt, Ws, bs,
           g1, b1, Wg, a_src, a_dst, We, a_e, bg, W_dec, b_dec):
    xf = x.reshape(N, 24)
    src = edge_index[0]
    dst = edge_index[1]
    npe = EP - E
    pad_src = jnp.arange(npe, dtype=jnp.int32) % N
    pad_dst = N + (jnp.arange(npe, dtype=jnp.int32) % (NPAD - N))
    srcP = jnp.concatenate([src, pad_src]).reshape(NS, CHUNKS, CH)
    dstP = jnp.concatenate([dst, pad_dst]).reshape(NS, CHUNKS, CH)
    eaT = jnp.concatenate([edge_attr.T, jnp.zeros((4, npe), F32)], axis=1)
    z64 = jnp.zeros((NPAD, HH), F32)
    z16 = jnp.zeros((NPAD, 16), F32)

    hw, sd = _prologue_call(
        xf, t, s, W_embed, b_embed.reshape(1, HID), Wt, bt.reshape(1, HID),
        Ws, bs.reshape(1, HID), g1.reshape(1, HID), b1.reshape(1, HID),
        Wg[0], jnp.stack([a_src[0], a_dst[0]], axis=1))
    eal, wea, easum = _edge_prologue_call(eaT, We, a_e.reshape(NL, HID, 1))
    ealP = eal.reshape(NL, NS, CHUNKS, CH)

    for i in range(NL):
        num, den = _sc_edge(hw, sd.reshape(2 * N), srcP, dstP, ealP[i],
                            z64, z16)
        weal = wea[:, i:i + 1]
        asum = (a_src[i] + a_dst[i]).reshape(HID, 1)
        if i < NL - 1:
            hw, sd = _combine_mid_call(
                num, den, hw, easum, weal, asum, bg[i].reshape(1, HID),
                Wg[i + 1], jnp.stack([a_src[i + 1], a_dst[i + 1]], axis=1))
        else:
            out = _combine_final_call(
                num, den, hw, easum, weal, asum, bg[i].reshape(1, HID),
                W_dec, b_dec.reshape(1, 24))
    return out.reshape(N, 2, 12)
